# baseline (speedup 1.0000x reference)
import functools

import jax
import jax.numpy as jnp
from jax.experimental import pallas as pl
from jax.experimental.pallas import tpu as pltpu

B = 2
N = 16384
NPOINT = 2048
NSAMPLE = 16
IN_CH = 16
OUT_CH = 16
D_FEAT = 3 * IN_CH - 3
EPS = 1e-6
NEG_SLOPE = 0.2
NROW = 128
NCOL = 128


SROW = NPOINT // NCOL


def _fps_body(xyz_ref, out_ref):
    x = xyz_ref[:, 0]
    y = xyz_ref[:, 1]
    z = xyz_ref[:, 2]
    row_io = jax.lax.broadcasted_iota(jnp.int32, (NROW, NCOL), 0)
    col_io = jax.lax.broadcasted_iota(jnp.int32, (NROW, NCOL), 1)
    flat_io = row_io * NCOL + col_io
    out_io = (jax.lax.broadcasted_iota(jnp.int32, (SROW, NCOL), 0) * NCOL
              + jax.lax.broadcasted_iota(jnp.int32, (SROW, NCOL), 1))
    lane_io = jax.lax.broadcasted_iota(jnp.int32, (1, NCOL), 1)
    init = jnp.full((NROW, NCOL), 1e10, jnp.float32)
    oinit = jnp.zeros((SROW, NCOL), jnp.int32)

    def step(i, carry):
        fars = carry[:B]
        dists = carry[B:2 * B]
        outs = carry[2 * B:]
        new_fars = []
        new_dists = []
        new_outs = []
        for b in range(B):
            far = fars[b]
            r = far // NCOL
            c = far - r * NCOL
            lane_sel = lane_io == c
            xr = xyz_ref[b, 0, pl.ds(r, 1), :]
            yr = xyz_ref[b, 1, pl.ds(r, 1), :]
            zr = xyz_ref[b, 2, pl.ds(r, 1), :]
            cx = jnp.sum(jnp.where(lane_sel, xr, 0.0))
            cy = jnp.sum(jnp.where(lane_sel, yr, 0.0))
            cz = jnp.sum(jnp.where(lane_sel, zr, 0.0))
            d = (x[b] - cx) ** 2 + (y[b] - cy) ** 2 + (z[b] - cz) ** 2
            db = jnp.minimum(dists[b], d)
            m = jnp.max(db)
            nxt = jnp.min(jnp.where(db == m, flat_io, jnp.int32(N)))
            new_fars.append(nxt)
            new_dists.append(db)
            new_outs.append(jnp.where(out_io == i, far, outs[b]))
        return tuple(new_fars) + tuple(new_dists) + tuple(new_outs)

    fin = jax.lax.fori_loop(
        0, NPOINT, step,
        (jnp.int32(0),) * B + (init,) * B + (oinit,) * B)
    for b in range(B):
        out_ref[b] = fin[2 * B + b]


def _fps_pallas(xyz):
    xyz4 = xyz.reshape(B, 3, NROW, NCOL)
    out = pl.pallas_call(
        _fps_body,
        out_shape=jax.ShapeDtypeStruct((B, SROW, NCOL), jnp.int32),
        in_specs=[pl.BlockSpec(memory_space=pltpu.VMEM)],
        out_specs=pl.BlockSpec(memory_space=pltpu.VMEM),
    )(xyz4)
    return out.reshape(B, NPOINT)


def _square_distance(src, dst):
    dist = -2.0 * jnp.matmul(src, dst.transpose(0, 2, 1))
    dist = dist + jnp.sum(src ** 2, -1)[:, :, None]
    dist = dist + jnp.sum(dst ** 2, -1)[:, None, :]
    return dist


def _vn_linear_leaky(x, W_feat, W_dir):
    p = jnp.einsum('oc,bcdsk->bodsk', W_feat, x)
    d = jnp.einsum('oc,bcdsk->bodsk', W_dir, x)
    dotprod = jnp.sum(p * d, axis=2, keepdims=True)
    mask = (dotprod >= 0).astype(x.dtype)
    d_norm_sq = jnp.sum(d * d, axis=2, keepdims=True)
    x_out = NEG_SLOPE * p + (1 - NEG_SLOPE) * (
        mask * p + (1 - mask) * (p - (dotprod / (d_norm_sq + EPS)) * d))
    return x_out


def _vn_max_pool(x, W_pool):
    d = jnp.einsum('oc,bcdsk->bodsk', W_pool, x)
    dotprod = jnp.sum(x * d, axis=2)
    idx = jnp.argmax(dotprod, axis=-1)
    x_max = jnp.take_along_axis(x, idx[:, :, None, :, None], axis=4)
    return x_max[..., 0]


def kernel(xyz, points, W_feat, W_dir, W_pool):
    xyz_t = xyz.transpose(0, 2, 1)
    pts_t = points.transpose(0, 2, 1)
    fps_idx = _fps_pallas(xyz)
    new_xyz = jax.vmap(lambda a, i: a[i])(xyz_t, fps_idx)
    sqrdists = _square_distance(new_xyz, xyz_t)
    _, idx = jax.lax.top_k(-sqrdists, NSAMPLE)
    return new_xyz.transpose(0, 2, 1), jnp.zeros((B, 48, NPOINT), jnp.float32) + idx[:, :, :1].transpose(0, 2, 1), fps_idx
    grouped_xyz = jax.vmap(lambda pts, ix: pts[ix])(xyz_t, idx)
    grouped_xyz_norm = grouped_xyz - new_xyz[:, :, None, :]
    grouped_points = jax.vmap(lambda pts, ix: pts[ix])(pts_t, idx)
    new_points = jnp.concatenate([grouped_xyz_norm, grouped_points], axis=-1)
    new_points = new_points.reshape(B, NPOINT, NSAMPLE, -1, 3).transpose(0, 3, 4, 1, 2)
    new_points = _vn_linear_leaky(new_points, W_feat, W_dir)
    new_points = _vn_max_pool(new_points, W_pool).reshape(B, -1, NPOINT)
    return new_xyz.transpose(0, 2, 1), new_points, fps_idx


# speedup vs baseline: 4.7646x; 4.7646x over previous
import functools

import jax
import jax.numpy as jnp
from jax.experimental import pallas as pl
from jax.experimental.pallas import tpu as pltpu

B = 2
N = 16384
NPOINT = 2048
NSAMPLE = 16
IN_CH = 16
OUT_CH = 16
D_FEAT = 3 * IN_CH - 3
EPS = 1e-6
NEG_SLOPE = 0.2
NROW = 128
NCOL = 128
NCHUNK = 128
CHUNK = N // NCHUNK
SROW = NPOINT // NCOL


def _fps_body(xyz_ref, out_ref):
    x = xyz_ref[:, 0]
    y = xyz_ref[:, 1]
    z = xyz_ref[:, 2]
    row_io = jax.lax.broadcasted_iota(jnp.int32, (NROW, NCOL), 0)
    col_io = jax.lax.broadcasted_iota(jnp.int32, (NROW, NCOL), 1)
    flat_io = row_io * NCOL + col_io
    out_io = (jax.lax.broadcasted_iota(jnp.int32, (SROW, NCOL), 0) * NCOL
              + jax.lax.broadcasted_iota(jnp.int32, (SROW, NCOL), 1))
    lane_io = jax.lax.broadcasted_iota(jnp.int32, (1, NCOL), 1)
    init = jnp.full((NROW, NCOL), 1e10, jnp.float32)
    oinit = jnp.zeros((SROW, NCOL), jnp.int32)

    def step(i, carry):
        fars = carry[:B]
        dists = carry[B:2 * B]
        outs = carry[2 * B:]
        new_fars = []
        new_dists = []
        new_outs = []
        for b in range(B):
            far = fars[b]
            r = far // NCOL
            c = far - r * NCOL
            lane_sel = lane_io == c
            xr = xyz_ref[b, 0, pl.ds(r, 1), :]
            yr = xyz_ref[b, 1, pl.ds(r, 1), :]
            zr = xyz_ref[b, 2, pl.ds(r, 1), :]
            cx = jnp.sum(jnp.where(lane_sel, xr, 0.0))
            cy = jnp.sum(jnp.where(lane_sel, yr, 0.0))
            cz = jnp.sum(jnp.where(lane_sel, zr, 0.0))
            d = (x[b] - cx) ** 2 + (y[b] - cy) ** 2 + (z[b] - cz) ** 2
            db = jnp.minimum(dists[b], d)
            m = jnp.max(db)
            nxt = jnp.min(jnp.where(db == m, flat_io, jnp.int32(N)))
            new_fars.append(nxt)
            new_dists.append(db)
            new_outs.append(jnp.where(out_io == i, far, outs[b]))
        return tuple(new_fars) + tuple(new_dists) + tuple(new_outs)

    fin = jax.lax.fori_loop(
        0, NPOINT, step,
        (jnp.int32(0),) * B + (init,) * B + (oinit,) * B)
    for b in range(B):
        out_ref[b] = fin[2 * B + b]


def _fps_pallas(xyz):
    xyz4 = xyz.reshape(B, 3, NROW, NCOL)
    out = pl.pallas_call(
        _fps_body,
        out_shape=jax.ShapeDtypeStruct((B, SROW, NCOL), jnp.int32),
        in_specs=[pl.BlockSpec(memory_space=pltpu.VMEM)],
        out_specs=pl.BlockSpec(memory_space=pltpu.VMEM),
    )(xyz4)
    return out.reshape(B, NPOINT)


TS_D = 128
TS_R = 512


def _dist_body(q_ref, x_ref, dist_ref, cidx_ref):
    q = q_ref[0]
    x = x_ref[0]
    qq = jnp.sum(q * q, axis=1, keepdims=True)
    xx = jnp.sum(x * x, axis=0, keepdims=True)
    d = -2.0 * jax.lax.dot(q, x, preferred_element_type=jnp.float32) + qq + xx
    dist_ref[0] = d
    cm = jnp.min(d.reshape(TS_D, NCHUNK, CHUNK), axis=2)
    chunk_io = jax.lax.broadcasted_iota(jnp.int32, (TS_D, NCHUNK), 1)
    col16 = jax.lax.broadcasted_iota(jnp.int32, (TS_D, NSAMPLE), 1)
    acc = jnp.zeros((TS_D, NSAMPLE), jnp.int32)
    for k in range(NSAMPLE):
        m = jnp.min(cm, axis=1, keepdims=True)
        ck = jnp.min(jnp.where(cm == m, chunk_io, NCHUNK), axis=1,
                     keepdims=True)
        acc = jnp.where(col16 == k, ck, acc)
        cm = jnp.where(chunk_io == ck, jnp.inf, cm)
    cidx_ref[0] = acc


def _refine_body(cand_ref, gmap_ref, idx_ref):
    c = cand_ref[0]
    g = gmap_ref[0]
    col16 = jax.lax.broadcasted_iota(jnp.int32, (TS_R, NSAMPLE), 1)
    acc = jnp.zeros((TS_R, NSAMPLE), jnp.int32)
    for k in range(NSAMPLE):
        m = jnp.min(c, axis=1, keepdims=True)
        sel = c == m
        gk = jnp.min(jnp.where(sel, g, N), axis=1, keepdims=True)
        acc = jnp.where(col16 == k, gk, acc)
        c = jnp.where(g == gk, jnp.inf, c)
    idx_ref[0] = acc


def _knn_pallas(new_xyz, xyz):
    S = NPOINT
    dist, cidx = pl.pallas_call(
        _dist_body,
        grid=(B, S // TS_D),
        in_specs=[
            pl.BlockSpec((1, TS_D, 3), lambda b, s: (b, s, 0)),
            pl.BlockSpec((1, 3, N), lambda b, s: (b, 0, 0)),
        ],
        out_specs=[
            pl.BlockSpec((1, TS_D, N), lambda b, s: (b, s, 0)),
            pl.BlockSpec((1, TS_D, NSAMPLE), lambda b, s: (b, s, 0)),
        ],
        out_shape=[
            jax.ShapeDtypeStruct((B, S, N), jnp.float32),
            jax.ShapeDtypeStruct((B, S, NSAMPLE), jnp.int32),
        ],
    )(new_xyz, xyz)
    dist4 = dist.reshape(B, S, NCHUNK, CHUNK)
    cand = jnp.take_along_axis(dist4, cidx[:, :, :, None], axis=2)
    gmap = cidx[:, :, :, None] * CHUNK + jnp.arange(CHUNK, dtype=jnp.int32)
    cand2 = cand.reshape(B, S, NSAMPLE * CHUNK)
    gmap2 = gmap.reshape(B, S, NSAMPLE * CHUNK)
    idx = pl.pallas_call(
        _refine_body,
        grid=(B, S // TS_R),
        in_specs=[
            pl.BlockSpec((1, TS_R, NSAMPLE * CHUNK), lambda b, s: (b, s, 0)),
            pl.BlockSpec((1, TS_R, NSAMPLE * CHUNK), lambda b, s: (b, s, 0)),
        ],
        out_specs=pl.BlockSpec((1, TS_R, NSAMPLE), lambda b, s: (b, s, 0)),
        out_shape=jax.ShapeDtypeStruct((B, S, NSAMPLE), jnp.int32),
    )(cand2, gmap2)
    return idx


def _vn_linear_leaky(x, W_feat, W_dir):
    p = jnp.einsum('oc,bcdsk->bodsk', W_feat, x)
    d = jnp.einsum('oc,bcdsk->bodsk', W_dir, x)
    dotprod = jnp.sum(p * d, axis=2, keepdims=True)
    mask = (dotprod >= 0).astype(x.dtype)
    d_norm_sq = jnp.sum(d * d, axis=2, keepdims=True)
    x_out = NEG_SLOPE * p + (1 - NEG_SLOPE) * (
        mask * p + (1 - mask) * (p - (dotprod / (d_norm_sq + EPS)) * d))
    return x_out


def _vn_max_pool(x, W_pool):
    d = jnp.einsum('oc,bcdsk->bodsk', W_pool, x)
    dotprod = jnp.sum(x * d, axis=2)
    idx = jnp.argmax(dotprod, axis=-1)
    x_max = jnp.take_along_axis(x, idx[:, :, None, :, None], axis=4)
    return x_max[..., 0]


def kernel(xyz, points, W_feat, W_dir, W_pool):
    xyz_t = xyz.transpose(0, 2, 1)
    pts_t = points.transpose(0, 2, 1)
    fps_idx = _fps_pallas(xyz)
    new_xyz = jax.vmap(lambda a, i: a[i])(xyz_t, fps_idx)
    idx = _knn_pallas(new_xyz, xyz)
    grouped_xyz = jax.vmap(lambda pts, ix: pts[ix])(xyz_t, idx)
    grouped_xyz_norm = grouped_xyz - new_xyz[:, :, None, :]
    grouped_points = jax.vmap(lambda pts, ix: pts[ix])(pts_t, idx)
    new_points = jnp.concatenate([grouped_xyz_norm, grouped_points], axis=-1)
    new_points = new_points.reshape(B, NPOINT, NSAMPLE, -1, 3).transpose(0, 3, 4, 1, 2)
    new_points = _vn_linear_leaky(new_points, W_feat, W_dir)
    new_points = _vn_max_pool(new_points, W_pool).reshape(B, -1, NPOINT)
    return new_xyz.transpose(0, 2, 1), new_points, fps_idx


# speedup vs baseline: 4.7780x; 1.0028x over previous
import functools

import jax
import jax.numpy as jnp
from jax.experimental import pallas as pl
from jax.experimental.pallas import tpu as pltpu

B = 2
N = 16384
NPOINT = 2048
NSAMPLE = 16
IN_CH = 16
OUT_CH = 16
D_FEAT = 3 * IN_CH - 3
EPS = 1e-6
NEG_SLOPE = 0.2
NROW = 128
NCOL = 128
NCHUNK = 128
CHUNK = N // NCHUNK
SROW = NPOINT // NCOL


def _fps_body(xyz_ref, out_ref):
    x = xyz_ref[:, 0]
    y = xyz_ref[:, 1]
    z = xyz_ref[:, 2]
    row_io = jax.lax.broadcasted_iota(jnp.int32, (NROW, NCOL), 0)
    col_io = jax.lax.broadcasted_iota(jnp.int32, (NROW, NCOL), 1)
    flat_io = row_io * NCOL + col_io
    out_io = (jax.lax.broadcasted_iota(jnp.int32, (SROW, NCOL), 0) * NCOL
              + jax.lax.broadcasted_iota(jnp.int32, (SROW, NCOL), 1))
    lane_io = jax.lax.broadcasted_iota(jnp.int32, (1, NCOL), 1)
    init = jnp.full((NROW, NCOL), 1e10, jnp.float32)
    oinit = jnp.zeros((SROW, NCOL), jnp.int32)

    def step(i, carry):
        fars = carry[:B]
        dists = carry[B:2 * B]
        outs = carry[2 * B:]
        new_fars = []
        new_dists = []
        new_outs = []
        for b in range(B):
            far = fars[b]
            r = far // NCOL
            c = far - r * NCOL
            lane_sel = lane_io == c
            xr = xyz_ref[b, 0, pl.ds(r, 1), :]
            yr = xyz_ref[b, 1, pl.ds(r, 1), :]
            zr = xyz_ref[b, 2, pl.ds(r, 1), :]
            cx = jnp.sum(jnp.where(lane_sel, xr, 0.0))
            cy = jnp.sum(jnp.where(lane_sel, yr, 0.0))
            cz = jnp.sum(jnp.where(lane_sel, zr, 0.0))
            d = (x[b] - cx) ** 2 + (y[b] - cy) ** 2 + (z[b] - cz) ** 2
            db = jnp.minimum(dists[b], d)
            m = jnp.max(db)
            nxt = jnp.min(jnp.where(db == m, flat_io, jnp.int32(N)))
            new_fars.append(nxt)
            new_dists.append(db)
            new_outs.append(jnp.where(out_io == i, far, outs[b]))
        return tuple(new_fars) + tuple(new_dists) + tuple(new_outs)

    fin = jax.lax.fori_loop(
        0, NPOINT, step,
        (jnp.int32(0),) * B + (init,) * B + (oinit,) * B)
    for b in range(B):
        out_ref[b] = fin[2 * B + b]


def _fps_pallas(xyz):
    xyz4 = xyz.reshape(B, 3, NROW, NCOL)
    out = pl.pallas_call(
        _fps_body,
        out_shape=jax.ShapeDtypeStruct((B, SROW, NCOL), jnp.int32),
        in_specs=[pl.BlockSpec(memory_space=pltpu.VMEM)],
        out_specs=pl.BlockSpec(memory_space=pltpu.VMEM),
    )(xyz4)
    return out.reshape(B, NPOINT)


TS_D = 128
TS_R = 512


def _dist_body(q_ref, x_ref, dist_ref, cidx_ref):
    q = q_ref[0]
    x = x_ref[0]
    qq = jnp.sum(q * q, axis=1, keepdims=True)
    xx = jnp.sum(x * x, axis=0, keepdims=True)
    d = -2.0 * jax.lax.dot(q, x, preferred_element_type=jnp.float32) + qq + xx
    dist_ref[0] = d
    cm = jnp.min(d.reshape(TS_D, NCHUNK, CHUNK), axis=2)
    chunk_io = jax.lax.broadcasted_iota(jnp.int32, (TS_D, NCHUNK), 1)
    col16 = jax.lax.broadcasted_iota(jnp.int32, (TS_D, NSAMPLE), 1)
    acc = jnp.zeros((TS_D, NSAMPLE), jnp.int32)
    for k in range(NSAMPLE):
        m = jnp.min(cm, axis=1, keepdims=True)
        ck = jnp.min(jnp.where(cm == m, chunk_io, NCHUNK), axis=1,
                     keepdims=True)
        acc = jnp.where(col16 == k, ck, acc)
        cm = jnp.where(chunk_io == ck, jnp.inf, cm)
    cidx_ref[0] = acc


def _refine_body(cand_ref, gmap_ref, idx_ref):
    c = cand_ref[0]
    g = gmap_ref[0]
    col16 = jax.lax.broadcasted_iota(jnp.int32, (TS_R, NSAMPLE), 1)
    acc = jnp.zeros((TS_R, NSAMPLE), jnp.int32)
    for k in range(NSAMPLE):
        m = jnp.min(c, axis=1, keepdims=True)
        sel = c == m
        gk = jnp.min(jnp.where(sel, g, N), axis=1, keepdims=True)
        acc = jnp.where(col16 == k, gk, acc)
        c = jnp.where(g == gk, jnp.inf, c)
    idx_ref[0] = acc


def _knn_pallas(new_xyz, xyz):
    S = NPOINT
    dist, cidx = pl.pallas_call(
        _dist_body,
        grid=(B, S // TS_D),
        in_specs=[
            pl.BlockSpec((1, TS_D, 3), lambda b, s: (b, s, 0)),
            pl.BlockSpec((1, 3, N), lambda b, s: (b, 0, 0)),
        ],
        out_specs=[
            pl.BlockSpec((1, TS_D, N), lambda b, s: (b, s, 0)),
            pl.BlockSpec((1, TS_D, NSAMPLE), lambda b, s: (b, s, 0)),
        ],
        out_shape=[
            jax.ShapeDtypeStruct((B, S, N), jnp.float32),
            jax.ShapeDtypeStruct((B, S, NSAMPLE), jnp.int32),
        ],
    )(new_xyz, xyz)
    dist4 = dist.reshape(B, S, NCHUNK, CHUNK)
    cand = jnp.take_along_axis(dist4, cidx[:, :, :, None], axis=2)
    gmap = cidx[:, :, :, None] * CHUNK + jnp.arange(CHUNK, dtype=jnp.int32)
    cand2 = cand.reshape(B, S, NSAMPLE * CHUNK)
    gmap2 = gmap.reshape(B, S, NSAMPLE * CHUNK)
    idx = pl.pallas_call(
        _refine_body,
        grid=(B, S // TS_R),
        in_specs=[
            pl.BlockSpec((1, TS_R, NSAMPLE * CHUNK), lambda b, s: (b, s, 0)),
            pl.BlockSpec((1, TS_R, NSAMPLE * CHUNK), lambda b, s: (b, s, 0)),
        ],
        out_specs=pl.BlockSpec((1, TS_R, NSAMPLE), lambda b, s: (b, s, 0)),
        out_shape=jax.ShapeDtypeStruct((B, S, NSAMPLE), jnp.int32),
    )(cand2, gmap2)
    return idx


TS_V = 512


def _vn_body(x_ref, wf_ref, wd_ref, wp_ref, out_ref):
    wf = wf_ref[...]
    wd = wd_ref[...]
    wp = wp_ref[...]
    xs = [x_ref[0, :, dd, :] for dd in range(3)]
    ps = [jax.lax.dot(wf, xs[dd], preferred_element_type=jnp.float32)
          for dd in range(3)]
    ds = [jax.lax.dot(wd, xs[dd], preferred_element_type=jnp.float32)
          for dd in range(3)]
    dotpd = ps[0] * ds[0] + ps[1] * ds[1] + ps[2] * ds[2]
    dnorm = ds[0] * ds[0] + ds[1] * ds[1] + ds[2] * ds[2]
    mask = dotpd >= 0
    coef = dotpd / (dnorm + EPS)
    xo = [NEG_SLOPE * ps[dd]
          + (1 - NEG_SLOPE) * jnp.where(mask, ps[dd], ps[dd] - coef * ds[dd])
          for dd in range(3)]
    pool = [jax.lax.dot(wp, xo[dd], preferred_element_type=jnp.float32)
            for dd in range(3)]
    dp = (xo[0] * pool[0] + xo[1] * pool[1] + xo[2] * pool[2]
          ).reshape(OUT_CH, TS_V, NSAMPLE)
    m = jnp.max(dp, axis=2, keepdims=True)
    kio = jax.lax.broadcasted_iota(jnp.int32, (OUT_CH, TS_V, NSAMPLE), 2)
    ksel = jnp.min(jnp.where(dp == m, kio, NSAMPLE), axis=2, keepdims=True)
    onehot = kio == ksel
    outs = [jnp.sum(jnp.where(onehot, xo[dd].reshape(OUT_CH, TS_V, NSAMPLE),
                              0.0), axis=2)
            for dd in range(3)]
    y = jnp.concatenate([o[:, None, :] for o in outs], axis=1)
    out_ref[0] = y.reshape(3 * OUT_CH, TS_V)


def _vn_pallas(x4, W_feat, W_dir, W_pool):
    S = NPOINT
    return pl.pallas_call(
        _vn_body,
        grid=(B, S // TS_V),
        in_specs=[
            pl.BlockSpec((1, IN_CH, 3, TS_V * NSAMPLE),
                         lambda b, s: (b, 0, 0, s)),
            pl.BlockSpec((OUT_CH, IN_CH), lambda b, s: (0, 0)),
            pl.BlockSpec((OUT_CH, IN_CH), lambda b, s: (0, 0)),
            pl.BlockSpec((OUT_CH, OUT_CH), lambda b, s: (0, 0)),
        ],
        out_specs=pl.BlockSpec((1, 3 * OUT_CH, TS_V), lambda b, s: (b, 0, s)),
        out_shape=jax.ShapeDtypeStruct((B, 3 * OUT_CH, S), jnp.float32),
    )(x4, W_feat, W_dir, W_pool)


def kernel(xyz, points, W_feat, W_dir, W_pool):
    xyz_t = xyz.transpose(0, 2, 1)
    pts_t = points.transpose(0, 2, 1)
    fps_idx = _fps_pallas(xyz)
    new_xyz = jax.vmap(lambda a, i: a[i])(xyz_t, fps_idx)
    idx = _knn_pallas(new_xyz, xyz)
    grouped_xyz = jax.vmap(lambda pts, ix: pts[ix])(xyz_t, idx)
    grouped_xyz_norm = grouped_xyz - new_xyz[:, :, None, :]
    grouped_points = jax.vmap(lambda pts, ix: pts[ix])(pts_t, idx)
    new_points = jnp.concatenate([grouped_xyz_norm, grouped_points], axis=-1)
    x4 = (new_points.reshape(B, NPOINT, NSAMPLE, IN_CH, 3)
          .transpose(0, 3, 4, 1, 2)
          .reshape(B, IN_CH, 3, NPOINT * NSAMPLE))
    new_points = _vn_pallas(x4, W_feat, W_dir, W_pool)
    return new_xyz.transpose(0, 2, 1), new_points, fps_idx


# speedup vs baseline: 4.8741x; 1.0201x over previous
import functools

import jax
import jax.numpy as jnp
from jax.experimental import pallas as pl
from jax.experimental.pallas import tpu as pltpu

B = 2
N = 16384
NPOINT = 2048
NSAMPLE = 16
IN_CH = 16
OUT_CH = 16
D_FEAT = 3 * IN_CH - 3
EPS = 1e-6
NEG_SLOPE = 0.2
NROW = 128
NCOL = 128
NCHUNK = 128
CHUNK = N // NCHUNK
SROW = NPOINT // NCOL


def _fps_body(xyz_ref, out_ref):
    x = xyz_ref[:, 0]
    y = xyz_ref[:, 1]
    z = xyz_ref[:, 2]
    row_io = jax.lax.broadcasted_iota(jnp.int32, (NROW, NCOL), 0)
    col_io = jax.lax.broadcasted_iota(jnp.int32, (NROW, NCOL), 1)
    flat_io = row_io * NCOL + col_io
    out_io = (jax.lax.broadcasted_iota(jnp.int32, (SROW, NCOL), 0) * NCOL
              + jax.lax.broadcasted_iota(jnp.int32, (SROW, NCOL), 1))
    lane_io = jax.lax.broadcasted_iota(jnp.int32, (1, NCOL), 1)
    init = jnp.full((NROW, NCOL), 1e10, jnp.float32)
    oinit = jnp.zeros((SROW, NCOL), jnp.int32)

    def step(i, carry):
        fars = carry[:B]
        dists = carry[B:2 * B]
        outs = carry[2 * B:]
        new_fars = []
        new_dists = []
        new_outs = []
        for b in range(B):
            far = fars[b]
            r = far // NCOL
            c = far - r * NCOL
            lane_sel = lane_io == c
            xr = xyz_ref[b, 0, pl.ds(r, 1), :]
            yr = xyz_ref[b, 1, pl.ds(r, 1), :]
            zr = xyz_ref[b, 2, pl.ds(r, 1), :]
            cx = jnp.sum(jnp.where(lane_sel, xr, 0.0))
            cy = jnp.sum(jnp.where(lane_sel, yr, 0.0))
            cz = jnp.sum(jnp.where(lane_sel, zr, 0.0))
            d = (x[b] - cx) ** 2 + (y[b] - cy) ** 2 + (z[b] - cz) ** 2
            db = jnp.minimum(dists[b], d)
            m = jnp.max(db)
            nxt = jnp.min(jnp.where(db == m, flat_io, jnp.int32(N)))
            new_fars.append(nxt)
            new_dists.append(db)
            new_outs.append(jnp.where(out_io == i, far, outs[b]))
        return tuple(new_fars) + tuple(new_dists) + tuple(new_outs)

    fin = jax.lax.fori_loop(
        0, NPOINT, step,
        (jnp.int32(0),) * B + (init,) * B + (oinit,) * B)
    for b in range(B):
        out_ref[b] = fin[2 * B + b]


def _fps_pallas(xyz):
    xyz4 = xyz.reshape(B, 3, NROW, NCOL)
    out = pl.pallas_call(
        _fps_body,
        out_shape=jax.ShapeDtypeStruct((B, SROW, NCOL), jnp.int32),
        in_specs=[pl.BlockSpec(memory_space=pltpu.VMEM)],
        out_specs=pl.BlockSpec(memory_space=pltpu.VMEM),
    )(xyz4)
    return out.reshape(B, NPOINT)


TS_D = 128
TS_R = 512


def _dist_body(q_ref, x_ref, dist_ref, cidx_ref):
    q = q_ref[0]
    x = x_ref[0]
    qq = jnp.sum(q * q, axis=1, keepdims=True)
    xx = jnp.sum(x * x, axis=0, keepdims=True)
    d = -2.0 * jax.lax.dot(q, x, preferred_element_type=jnp.float32) + qq + xx
    dist_ref[0] = d
    cm = jnp.min(d.reshape(TS_D, NCHUNK, CHUNK), axis=2)
    chunk_io = jax.lax.broadcasted_iota(jnp.int32, (TS_D, NCHUNK), 1)
    col16 = jax.lax.broadcasted_iota(jnp.int32, (TS_D, NSAMPLE), 1)
    acc = jnp.zeros((TS_D, NSAMPLE), jnp.int32)
    for k in range(NSAMPLE):
        m = jnp.min(cm, axis=1, keepdims=True)
        ck = jnp.min(jnp.where(cm == m, chunk_io, NCHUNK), axis=1,
                     keepdims=True)
        acc = jnp.where(col16 == k, ck, acc)
        cm = jnp.where(chunk_io == ck, jnp.inf, cm)
    cidx_ref[0] = acc


def _refine_body(cand_ref, gmap_ref, idx_ref):
    c = cand_ref[0]
    g = gmap_ref[0]
    col16 = jax.lax.broadcasted_iota(jnp.int32, (TS_R, NSAMPLE), 1)
    acc = jnp.zeros((TS_R, NSAMPLE), jnp.int32)
    for k in range(NSAMPLE):
        m = jnp.min(c, axis=1, keepdims=True)
        sel = c == m
        gk = jnp.min(jnp.where(sel, g, N), axis=1, keepdims=True)
        acc = jnp.where(col16 == k, gk, acc)
        c = jnp.where(g == gk, jnp.inf, c)
    idx_ref[0] = acc


def _knn_pallas(new_xyz, xyz):
    S = NPOINT
    dist, cidx = pl.pallas_call(
        _dist_body,
        grid=(B, S // TS_D),
        in_specs=[
            pl.BlockSpec((1, TS_D, 3), lambda b, s: (b, s, 0)),
            pl.BlockSpec((1, 3, N), lambda b, s: (b, 0, 0)),
        ],
        out_specs=[
            pl.BlockSpec((1, TS_D, N), lambda b, s: (b, s, 0)),
            pl.BlockSpec((1, TS_D, NSAMPLE), lambda b, s: (b, s, 0)),
        ],
        out_shape=[
            jax.ShapeDtypeStruct((B, S, N), jnp.float32),
            jax.ShapeDtypeStruct((B, S, NSAMPLE), jnp.int32),
        ],
    )(new_xyz, xyz)
    dist4 = dist.reshape(B, S, NCHUNK, CHUNK)
    cand = jnp.take_along_axis(dist4, cidx[:, :, :, None], axis=2)
    gmap = cidx[:, :, :, None] * CHUNK + jnp.arange(CHUNK, dtype=jnp.int32)
    cand2 = cand.reshape(B, S, NSAMPLE * CHUNK)
    gmap2 = gmap.reshape(B, S, NSAMPLE * CHUNK)
    idx = pl.pallas_call(
        _refine_body,
        grid=(B, S // TS_R),
        in_specs=[
            pl.BlockSpec((1, TS_R, NSAMPLE * CHUNK), lambda b, s: (b, s, 0)),
            pl.BlockSpec((1, TS_R, NSAMPLE * CHUNK), lambda b, s: (b, s, 0)),
        ],
        out_specs=pl.BlockSpec((1, TS_R, NSAMPLE), lambda b, s: (b, s, 0)),
        out_shape=jax.ShapeDtypeStruct((B, S, NSAMPLE), jnp.int32),
    )(cand2, gmap2)
    return idx


TS_V = 512


def _vn_body(gx_ref, gp_ref, q_ref, wfd_ref, wdd_ref, wp_ref, out_ref):
    wp = wp_ref[...]
    gx = gx_ref[0]
    gp = gp_ref[0]
    q = q_ref[0]
    qe = jnp.broadcast_to(q[:, None, :], (TS_V, NSAMPLE, 3)
                          ).reshape(TS_V * NSAMPLE, 3)
    g48 = jnp.concatenate([gx - qe, gp], axis=1)
    dn = (((1,), (1,)), ((), ()))
    ps = [jax.lax.dot_general(wfd_ref[dd], g48, dn,
                              preferred_element_type=jnp.float32)
          for dd in range(3)]
    ds = [jax.lax.dot_general(wdd_ref[dd], g48, dn,
                              preferred_element_type=jnp.float32)
          for dd in range(3)]
    dotpd = ps[0] * ds[0] + ps[1] * ds[1] + ps[2] * ds[2]
    dnorm = ds[0] * ds[0] + ds[1] * ds[1] + ds[2] * ds[2]
    mask = dotpd >= 0
    coef = dotpd / (dnorm + EPS)
    xo = [NEG_SLOPE * ps[dd]
          + (1 - NEG_SLOPE) * jnp.where(mask, ps[dd], ps[dd] - coef * ds[dd])
          for dd in range(3)]
    pool = [jax.lax.dot(wp, xo[dd], preferred_element_type=jnp.float32)
            for dd in range(3)]
    dp = (xo[0] * pool[0] + xo[1] * pool[1] + xo[2] * pool[2]
          ).reshape(OUT_CH, TS_V, NSAMPLE)
    m = jnp.max(dp, axis=2, keepdims=True)
    kio = jax.lax.broadcasted_iota(jnp.int32, (OUT_CH, TS_V, NSAMPLE), 2)
    ksel = jnp.min(jnp.where(dp == m, kio, NSAMPLE), axis=2, keepdims=True)
    onehot = kio == ksel
    outs = [jnp.sum(jnp.where(onehot, xo[dd].reshape(OUT_CH, TS_V, NSAMPLE),
                              0.0), axis=2)
            for dd in range(3)]
    y = jnp.concatenate([o[:, None, :] for o in outs], axis=1)
    out_ref[0] = y.reshape(3 * OUT_CH, TS_V)


def _vn_pallas(gx, gp, new_xyz, W_feat, W_dir, W_pool):
    S = NPOINT
    f = jnp.arange(3 * IN_CH)
    wfd = jnp.stack([W_feat[:, f // 3] * (f % 3 == dd) for dd in range(3)])
    wdd = jnp.stack([W_dir[:, f // 3] * (f % 3 == dd) for dd in range(3)])
    return pl.pallas_call(
        _vn_body,
        grid=(B, S // TS_V),
        in_specs=[
            pl.BlockSpec((1, TS_V * NSAMPLE, 3), lambda b, s: (b, s, 0)),
            pl.BlockSpec((1, TS_V * NSAMPLE, D_FEAT), lambda b, s: (b, s, 0)),
            pl.BlockSpec((1, TS_V, 3), lambda b, s: (b, s, 0)),
            pl.BlockSpec((3, OUT_CH, 3 * IN_CH), lambda b, s: (0, 0, 0)),
            pl.BlockSpec((3, OUT_CH, 3 * IN_CH), lambda b, s: (0, 0, 0)),
            pl.BlockSpec((OUT_CH, OUT_CH), lambda b, s: (0, 0)),
        ],
        out_specs=pl.BlockSpec((1, 3 * OUT_CH, TS_V), lambda b, s: (b, 0, s)),
        out_shape=jax.ShapeDtypeStruct((B, 3 * OUT_CH, S), jnp.float32),
    )(gx, gp, new_xyz, wfd, wdd, W_pool)


def kernel(xyz, points, W_feat, W_dir, W_pool):
    xyz_t = xyz.transpose(0, 2, 1)
    pts_t = points.transpose(0, 2, 1)
    fps_idx = _fps_pallas(xyz)
    new_xyz = jax.vmap(lambda a, i: a[i])(xyz_t, fps_idx)
    idx = _knn_pallas(new_xyz, xyz)
    idxf = idx.reshape(B, NPOINT * NSAMPLE)
    gx = jax.vmap(lambda p, i: p[i])(xyz_t, idxf)
    gp = jax.vmap(lambda p, i: p[i])(pts_t, idxf)
    new_points = _vn_pallas(gx, gp, new_xyz, W_feat, W_dir, W_pool)
    return new_xyz.transpose(0, 2, 1), new_points, fps_idx


# speedup vs baseline: 8.8997x; 1.8259x over previous
import functools

import jax
import jax.numpy as jnp
from jax import lax
from jax.experimental import pallas as pl
from jax.experimental.pallas import tpu as pltpu
from jax.experimental.pallas import tpu_sc as plsc

B = 2
N = 16384
NPOINT = 2048
NSAMPLE = 16
IN_CH = 16
OUT_CH = 16
D_FEAT = 3 * IN_CH - 3
EPS = 1e-6
NEG_SLOPE = 0.2
NROW = 128
NCOL = 128
NCHUNK = 128
CHUNK = N // NCHUNK
SROW = NPOINT // NCOL


def _fps_body(xyz_ref, out_ref):
    x = xyz_ref[:, 0]
    y = xyz_ref[:, 1]
    z = xyz_ref[:, 2]
    row_io = jax.lax.broadcasted_iota(jnp.int32, (NROW, NCOL), 0)
    col_io = jax.lax.broadcasted_iota(jnp.int32, (NROW, NCOL), 1)
    flat_io = row_io * NCOL + col_io
    out_io = (jax.lax.broadcasted_iota(jnp.int32, (SROW, NCOL), 0) * NCOL
              + jax.lax.broadcasted_iota(jnp.int32, (SROW, NCOL), 1))
    lane_io = jax.lax.broadcasted_iota(jnp.int32, (1, NCOL), 1)
    init = jnp.full((NROW, NCOL), 1e10, jnp.float32)
    oinit = jnp.zeros((SROW, NCOL), jnp.int32)

    def step(i, carry):
        fars = carry[:B]
        dists = carry[B:2 * B]
        outs = carry[2 * B:]
        new_fars = []
        new_dists = []
        new_outs = []
        for b in range(B):
            far = fars[b]
            r = far // NCOL
            c = far - r * NCOL
            lane_sel = lane_io == c
            xr = xyz_ref[b, 0, pl.ds(r, 1), :]
            yr = xyz_ref[b, 1, pl.ds(r, 1), :]
            zr = xyz_ref[b, 2, pl.ds(r, 1), :]
            cx = jnp.sum(jnp.where(lane_sel, xr, 0.0))
            cy = jnp.sum(jnp.where(lane_sel, yr, 0.0))
            cz = jnp.sum(jnp.where(lane_sel, zr, 0.0))
            d = (x[b] - cx) ** 2 + (y[b] - cy) ** 2 + (z[b] - cz) ** 2
            db = jnp.minimum(dists[b], d)
            m = jnp.max(db)
            nxt = jnp.min(jnp.where(db == m, flat_io, jnp.int32(N)))
            new_fars.append(nxt)
            new_dists.append(db)
            new_outs.append(jnp.where(out_io == i, far, outs[b]))
        return tuple(new_fars) + tuple(new_dists) + tuple(new_outs)

    fin = jax.lax.fori_loop(
        0, NPOINT, step,
        (jnp.int32(0),) * B + (init,) * B + (oinit,) * B)
    for b in range(B):
        out_ref[b] = fin[2 * B + b]


def _fps_pallas(xyz):
    xyz4 = xyz.reshape(B, 3, NROW, NCOL)
    out = pl.pallas_call(
        _fps_body,
        out_shape=jax.ShapeDtypeStruct((B, SROW, NCOL), jnp.int32),
        in_specs=[pl.BlockSpec(memory_space=pltpu.VMEM)],
        out_specs=pl.BlockSpec(memory_space=pltpu.VMEM),
    )(xyz4)
    return out.reshape(B, NPOINT)


TS_D = 128
TS_R = 512


def _dist_body(q_ref, x_ref, dist_ref, cidx_ref):
    q = q_ref[0]
    x = x_ref[0]
    qq = jnp.sum(q * q, axis=1, keepdims=True)
    xx = jnp.sum(x * x, axis=0, keepdims=True)
    d = -2.0 * jax.lax.dot(q, x, preferred_element_type=jnp.float32) + qq + xx
    dist_ref[0] = d
    cm = jnp.min(d.reshape(TS_D, NCHUNK, CHUNK), axis=2)
    chunk_io = jax.lax.broadcasted_iota(jnp.int32, (TS_D, NCHUNK), 1)
    col16 = jax.lax.broadcasted_iota(jnp.int32, (TS_D, NSAMPLE), 1)
    acc = jnp.zeros((TS_D, NSAMPLE), jnp.int32)
    for k in range(NSAMPLE):
        m = jnp.min(cm, axis=1, keepdims=True)
        ck = jnp.min(jnp.where(cm == m, chunk_io, NCHUNK), axis=1,
                     keepdims=True)
        acc = jnp.where(col16 == k, ck, acc)
        cm = jnp.where(chunk_io == ck, jnp.inf, cm)
    cidx_ref[0] = acc


def _refine_body(cand_ref, gmap_ref, idx_ref):
    c = cand_ref[0]
    g = gmap_ref[0]
    col16 = jax.lax.broadcasted_iota(jnp.int32, (TS_R, NSAMPLE), 1)
    acc = jnp.zeros((TS_R, NSAMPLE), jnp.int32)
    for k in range(NSAMPLE):
        m = jnp.min(c, axis=1, keepdims=True)
        sel = c == m
        gk = jnp.min(jnp.where(sel, g, N), axis=1, keepdims=True)
        acc = jnp.where(col16 == k, gk, acc)
        c = jnp.where(g == gk, jnp.inf, c)
    idx_ref[0] = acc


def _knn_pallas(new_xyz, xyz):
    S = NPOINT
    dist, cidx = pl.pallas_call(
        _dist_body,
        grid=(B, S // TS_D),
        in_specs=[
            pl.BlockSpec((1, TS_D, 3), lambda b, s: (b, s, 0)),
            pl.BlockSpec((1, 3, N), lambda b, s: (b, 0, 0)),
        ],
        out_specs=[
            pl.BlockSpec((1, TS_D, N), lambda b, s: (b, s, 0)),
            pl.BlockSpec((1, TS_D, NSAMPLE), lambda b, s: (b, s, 0)),
        ],
        out_shape=[
            jax.ShapeDtypeStruct((B, S, N), jnp.float32),
            jax.ShapeDtypeStruct((B, S, NSAMPLE), jnp.int32),
        ],
    )(new_xyz, xyz)
    dist4 = dist.reshape(B, S, NCHUNK, CHUNK)
    cand = jnp.take_along_axis(dist4, cidx[:, :, :, None], axis=2)
    gmap = cidx[:, :, :, None] * CHUNK + jnp.arange(CHUNK, dtype=jnp.int32)
    cand2 = cand.reshape(B, S, NSAMPLE * CHUNK)
    gmap2 = gmap.reshape(B, S, NSAMPLE * CHUNK)
    idx = pl.pallas_call(
        _refine_body,
        grid=(B, S // TS_R),
        in_specs=[
            pl.BlockSpec((1, TS_R, NSAMPLE * CHUNK), lambda b, s: (b, s, 0)),
            pl.BlockSpec((1, TS_R, NSAMPLE * CHUNK), lambda b, s: (b, s, 0)),
        ],
        out_specs=pl.BlockSpec((1, TS_R, NSAMPLE), lambda b, s: (b, s, 0)),
        out_shape=jax.ShapeDtypeStruct((B, S, NSAMPLE), jnp.int32),
    )(cand2, gmap2)
    return idx


NFEAT = 3 + D_FEAT
NFP = 128
M_G = B * NPOINT * NSAMPLE
NW_G = 32
PER_W = M_G // NW_G
CH_G = 128
NCHG = PER_W // CH_G
NB_G = 4


def _sc_gather(table, idx2):
    mesh = plsc.VectorSubcoreMesh(core_axis_name="c", subcore_axis_name="s")

    @functools.partial(
        pl.kernel, mesh=mesh,
        out_type=jax.ShapeDtypeStruct((M_G, NFP), jnp.float32),
        scratch_types=[
            pltpu.VMEM((NCHG, CH_G), jnp.int32),
            pltpu.VMEM((NB_G, CH_G, NFP), jnp.float32),
            pltpu.SemaphoreType.DMA,
        ],
    )
    def k(table_hbm, idx_hbm, out_hbm, idx_v, rows_v, sem):
        wid = lax.axis_index("s") * 2 + lax.axis_index("c")
        base = wid * PER_W
        pltpu.sync_copy(idx_hbm.at[pl.ds(wid * NCHG, NCHG)], idx_v)
        handles = [None] * NB_G
        for j in range(NCHG):
            bslot = j % NB_G
            if handles[bslot] is not None:
                handles[bslot].wait()
                pltpu.sync_copy(
                    rows_v.at[bslot],
                    out_hbm.at[pl.ds(base + (j - NB_G) * CH_G, CH_G)])
            handles[bslot] = pltpu.async_copy(
                table_hbm.at[idx_v.at[j]], rows_v.at[bslot], sem)
        for j in range(NCHG - NB_G, NCHG):
            bslot = j % NB_G
            handles[bslot].wait()
            pltpu.sync_copy(rows_v.at[bslot],
                            out_hbm.at[pl.ds(base + j * CH_G, CH_G)])

    return k(table, idx2)


TS_V = 512


def _vn_body(g_ref, q_ref, wfd_ref, wdd_ref, wp_ref, out_ref):
    wp = wp_ref[...]
    g0 = g_ref[0]
    q = q_ref[0]
    qe = jnp.broadcast_to(q[:, None, :], (TS_V, NSAMPLE, 3)
                          ).reshape(TS_V * NSAMPLE, 3)
    qe48 = jnp.concatenate(
        [qe, jnp.zeros((TS_V * NSAMPLE, NFP - 3), jnp.float32)], axis=1)
    g48 = g0 - qe48
    dn = (((1,), (1,)), ((), ()))
    ps = [jax.lax.dot_general(wfd_ref[dd], g48, dn,
                              preferred_element_type=jnp.float32)
          for dd in range(3)]
    ds = [jax.lax.dot_general(wdd_ref[dd], g48, dn,
                              preferred_element_type=jnp.float32)
          for dd in range(3)]
    dotpd = ps[0] * ds[0] + ps[1] * ds[1] + ps[2] * ds[2]
    dnorm = ds[0] * ds[0] + ds[1] * ds[1] + ds[2] * ds[2]
    mask = dotpd >= 0
    coef = dotpd / (dnorm + EPS)
    xo = [NEG_SLOPE * ps[dd]
          + (1 - NEG_SLOPE) * jnp.where(mask, ps[dd], ps[dd] - coef * ds[dd])
          for dd in range(3)]
    pool = [jax.lax.dot(wp, xo[dd], preferred_element_type=jnp.float32)
            for dd in range(3)]
    dp = (xo[0] * pool[0] + xo[1] * pool[1] + xo[2] * pool[2]
          ).reshape(OUT_CH, TS_V, NSAMPLE)
    m = jnp.max(dp, axis=2, keepdims=True)
    kio = jax.lax.broadcasted_iota(jnp.int32, (OUT_CH, TS_V, NSAMPLE), 2)
    ksel = jnp.min(jnp.where(dp == m, kio, NSAMPLE), axis=2, keepdims=True)
    onehot = kio == ksel
    outs = [jnp.sum(jnp.where(onehot, xo[dd].reshape(OUT_CH, TS_V, NSAMPLE),
                              0.0), axis=2)
            for dd in range(3)]
    y = jnp.concatenate([o[:, None, :] for o in outs], axis=1)
    out_ref[0] = y.reshape(3 * OUT_CH, TS_V)


def _vn_pallas(g48, new_xyz, W_feat, W_dir, W_pool):
    S = NPOINT
    f = jnp.arange(3 * IN_CH)
    wfd = jnp.stack([W_feat[:, f // 3] * (f % 3 == dd) for dd in range(3)])
    wdd = jnp.stack([W_dir[:, f // 3] * (f % 3 == dd) for dd in range(3)])
    wfd = jnp.pad(wfd, ((0, 0), (0, 0), (0, NFP - 3 * IN_CH)))
    wdd = jnp.pad(wdd, ((0, 0), (0, 0), (0, NFP - 3 * IN_CH)))
    return pl.pallas_call(
        _vn_body,
        grid=(B, S // TS_V),
        in_specs=[
            pl.BlockSpec((1, TS_V * NSAMPLE, NFP), lambda b, s: (b, s, 0)),
            pl.BlockSpec((1, TS_V, 3), lambda b, s: (b, s, 0)),
            pl.BlockSpec((3, OUT_CH, NFP), lambda b, s: (0, 0, 0)),
            pl.BlockSpec((3, OUT_CH, NFP), lambda b, s: (0, 0, 0)),
            pl.BlockSpec((OUT_CH, OUT_CH), lambda b, s: (0, 0)),
        ],
        out_specs=pl.BlockSpec((1, 3 * OUT_CH, TS_V), lambda b, s: (b, 0, s)),
        out_shape=jax.ShapeDtypeStruct((B, 3 * OUT_CH, S), jnp.float32),
    )(g48, new_xyz, wfd, wdd, W_pool)


def kernel(xyz, points, W_feat, W_dir, W_pool):
    xyz_t = xyz.transpose(0, 2, 1)
    pts_t = points.transpose(0, 2, 1)
    fps_idx = _fps_pallas(xyz)
    new_xyz = jax.vmap(lambda a, i: a[i])(xyz_t, fps_idx)
    idx = _knn_pallas(new_xyz, xyz)
    tbl = jnp.pad(jnp.concatenate([xyz_t, pts_t], axis=2),
                  ((0, 0), (0, 0), (0, NFP - NFEAT))).reshape(B * N, NFP)
    idx_g = (idx.reshape(B, NPOINT * NSAMPLE)
             + jnp.arange(B, dtype=jnp.int32)[:, None] * N)
    g48 = _sc_gather(tbl, idx_g.reshape(M_G // CH_G, CH_G))
    new_points = _vn_pallas(g48.reshape(B, NPOINT * NSAMPLE, NFP),
                            new_xyz, W_feat, W_dir, W_pool)
    return new_xyz.transpose(0, 2, 1), new_points, fps_idx


# speedup vs baseline: 12.6807x; 1.4249x over previous
import functools

import jax
import jax.numpy as jnp
from jax import lax
from jax.experimental import pallas as pl
from jax.experimental.pallas import tpu as pltpu
from jax.experimental.pallas import tpu_sc as plsc

B = 2
N = 16384
NPOINT = 2048
NSAMPLE = 16
IN_CH = 16
OUT_CH = 16
D_FEAT = 3 * IN_CH - 3
EPS = 1e-6
NEG_SLOPE = 0.2
NROW = 128
NCOL = 128
NCHUNK = 128
CHUNK = N // NCHUNK
SROW = NPOINT // NCOL


def _fps_body(xyz_ref, out_ref):
    x = xyz_ref[:, 0]
    y = xyz_ref[:, 1]
    z = xyz_ref[:, 2]
    flat3 = (jax.lax.broadcasted_iota(jnp.int32, (1, NROW, NCOL), 1) * NCOL
             + jax.lax.broadcasted_iota(jnp.int32, (1, NROW, NCOL), 2))
    out_io3 = (jax.lax.broadcasted_iota(jnp.int32, (1, SROW, NCOL), 1) * NCOL
               + jax.lax.broadcasted_iota(jnp.int32, (1, SROW, NCOL), 2))
    dists0 = jnp.full((B, NROW, NCOL), 1e10, jnp.float32)
    outs0 = jnp.zeros((B, SROW, NCOL), jnp.int32)
    far0 = jnp.zeros((B, 1, 1), jnp.int32)

    def step(i, carry):
        far, dists, outs = carry
        sel = flat3 == far
        cx = jnp.sum(jnp.where(sel, x, 0.0), axis=(1, 2), keepdims=True)
        cy = jnp.sum(jnp.where(sel, y, 0.0), axis=(1, 2), keepdims=True)
        cz = jnp.sum(jnp.where(sel, z, 0.0), axis=(1, 2), keepdims=True)
        d = (x - cx) ** 2 + (y - cy) ** 2 + (z - cz) ** 2
        db = jnp.minimum(dists, d)
        m = jnp.max(db, axis=(1, 2), keepdims=True)
        nxt = jnp.min(jnp.where(db == m, flat3, jnp.int32(N)),
                      axis=(1, 2), keepdims=True)
        outs = jnp.where(out_io3 == i, far, outs)
        return nxt, db, outs

    _, _, outs = jax.lax.fori_loop(0, NPOINT, step, (far0, dists0, outs0))
    out_ref[...] = outs


def _fps_pallas(xyz):
    xyz4 = xyz.reshape(B, 3, NROW, NCOL)
    out = pl.pallas_call(
        _fps_body,
        out_shape=jax.ShapeDtypeStruct((B, SROW, NCOL), jnp.int32),
        in_specs=[pl.BlockSpec(memory_space=pltpu.VMEM)],
        out_specs=pl.BlockSpec(memory_space=pltpu.VMEM),
    )(xyz4)
    return out.reshape(B, NPOINT)


TS_D = 128
TS_R = 512


def _dist_body(q_ref, x_ref, dist_ref, cidx_ref):
    q = q_ref[0]
    x = x_ref[0]
    qq = jnp.sum(q * q, axis=1, keepdims=True)
    xx = jnp.sum(x * x, axis=0, keepdims=True)
    d = -2.0 * jax.lax.dot(q, x, preferred_element_type=jnp.float32) + qq + xx
    dist_ref[0] = d
    cm = jnp.min(d.reshape(TS_D, NCHUNK, CHUNK), axis=2)
    chunk_io = jax.lax.broadcasted_iota(jnp.int32, (TS_D, NCHUNK), 1)
    col16 = jax.lax.broadcasted_iota(jnp.int32, (TS_D, NSAMPLE), 1)
    acc = jnp.zeros((TS_D, NSAMPLE), jnp.int32)
    for k in range(NSAMPLE):
        m = jnp.min(cm, axis=1, keepdims=True)
        ck = jnp.min(jnp.where(cm == m, chunk_io, NCHUNK), axis=1,
                     keepdims=True)
        acc = jnp.where(col16 == k, ck, acc)
        cm = jnp.where(chunk_io == ck, jnp.inf, cm)
    cidx_ref[0] = acc


def _refine_body(cand_ref, gmap_ref, idx_ref):
    c = cand_ref[0]
    g = gmap_ref[0]
    col16 = jax.lax.broadcasted_iota(jnp.int32, (TS_R, NSAMPLE), 1)
    acc = jnp.zeros((TS_R, NSAMPLE), jnp.int32)
    for k in range(NSAMPLE):
        m = jnp.min(c, axis=1, keepdims=True)
        sel = c == m
        gk = jnp.min(jnp.where(sel, g, N), axis=1, keepdims=True)
        acc = jnp.where(col16 == k, gk, acc)
        c = jnp.where(g == gk, jnp.inf, c)
    idx_ref[0] = acc


def _knn_pallas(new_xyz, xyz):
    S = NPOINT
    dist, cidx = pl.pallas_call(
        _dist_body,
        grid=(B, S // TS_D),
        in_specs=[
            pl.BlockSpec((1, TS_D, 3), lambda b, s: (b, s, 0)),
            pl.BlockSpec((1, 3, N), lambda b, s: (b, 0, 0)),
        ],
        out_specs=[
            pl.BlockSpec((1, TS_D, N), lambda b, s: (b, s, 0)),
            pl.BlockSpec((1, TS_D, NSAMPLE), lambda b, s: (b, s, 0)),
        ],
        out_shape=[
            jax.ShapeDtypeStruct((B, S, N), jnp.float32),
            jax.ShapeDtypeStruct((B, S, NSAMPLE), jnp.int32),
        ],
    )(new_xyz, xyz)
    dist4 = dist.reshape(B, S, NCHUNK, CHUNK)
    cand = jnp.take_along_axis(dist4, cidx[:, :, :, None], axis=2)
    gmap = cidx[:, :, :, None] * CHUNK + jnp.arange(CHUNK, dtype=jnp.int32)
    cand2 = cand.reshape(B, S, NSAMPLE * CHUNK)
    gmap2 = gmap.reshape(B, S, NSAMPLE * CHUNK)
    idx = pl.pallas_call(
        _refine_body,
        grid=(B, S // TS_R),
        in_specs=[
            pl.BlockSpec((1, TS_R, NSAMPLE * CHUNK), lambda b, s: (b, s, 0)),
            pl.BlockSpec((1, TS_R, NSAMPLE * CHUNK), lambda b, s: (b, s, 0)),
        ],
        out_specs=pl.BlockSpec((1, TS_R, NSAMPLE), lambda b, s: (b, s, 0)),
        out_shape=jax.ShapeDtypeStruct((B, S, NSAMPLE), jnp.int32),
    )(cand2, gmap2)
    return idx


NFEAT = 3 + D_FEAT
NFP = 128
M_G = B * NPOINT * NSAMPLE
NW_G = 32
PER_W = M_G // NW_G
CH_G = 128
NCHG = PER_W // CH_G
NB_G = 4


def _sc_gather(table, idx2):
    mesh = plsc.VectorSubcoreMesh(core_axis_name="c", subcore_axis_name="s")

    @functools.partial(
        pl.kernel, mesh=mesh,
        out_type=jax.ShapeDtypeStruct((M_G, NFP), jnp.float32),
        scratch_types=[
            pltpu.VMEM((NCHG, CH_G), jnp.int32),
            pltpu.VMEM((NB_G, CH_G, NFP), jnp.float32),
            pltpu.SemaphoreType.DMA,
        ],
    )
    def k(table_hbm, idx_hbm, out_hbm, idx_v, rows_v, sem):
        wid = lax.axis_index("s") * 2 + lax.axis_index("c")
        base = wid * PER_W
        pltpu.sync_copy(idx_hbm.at[pl.ds(wid * NCHG, NCHG)], idx_v)
        handles = [None] * NB_G
        for j in range(NCHG):
            bslot = j % NB_G
            if handles[bslot] is not None:
                handles[bslot].wait()
                pltpu.sync_copy(
                    rows_v.at[bslot],
                    out_hbm.at[pl.ds(base + (j - NB_G) * CH_G, CH_G)])
            handles[bslot] = pltpu.async_copy(
                table_hbm.at[idx_v.at[j]], rows_v.at[bslot], sem)
        for j in range(NCHG - NB_G, NCHG):
            bslot = j % NB_G
            handles[bslot].wait()
            pltpu.sync_copy(rows_v.at[bslot],
                            out_hbm.at[pl.ds(base + j * CH_G, CH_G)])

    return k(table, idx2)


TS_V = 512


def _vn_body(g_ref, q_ref, wfd_ref, wdd_ref, wp_ref, out_ref):
    wp = wp_ref[...]
    g0 = g_ref[0]
    q = q_ref[0]
    qe = jnp.broadcast_to(q[:, None, :], (TS_V, NSAMPLE, 3)
                          ).reshape(TS_V * NSAMPLE, 3)
    qe48 = jnp.concatenate(
        [qe, jnp.zeros((TS_V * NSAMPLE, NFP - 3), jnp.float32)], axis=1)
    g48 = g0 - qe48
    dn = (((1,), (1,)), ((), ()))
    ps = [jax.lax.dot_general(wfd_ref[dd], g48, dn,
                              preferred_element_type=jnp.float32)
          for dd in range(3)]
    ds = [jax.lax.dot_general(wdd_ref[dd], g48, dn,
                              preferred_element_type=jnp.float32)
          for dd in range(3)]
    dotpd = ps[0] * ds[0] + ps[1] * ds[1] + ps[2] * ds[2]
    dnorm = ds[0] * ds[0] + ds[1] * ds[1] + ds[2] * ds[2]
    mask = dotpd >= 0
    coef = dotpd / (dnorm + EPS)
    xo = [NEG_SLOPE * ps[dd]
          + (1 - NEG_SLOPE) * jnp.where(mask, ps[dd], ps[dd] - coef * ds[dd])
          for dd in range(3)]
    pool = [jax.lax.dot(wp, xo[dd], preferred_element_type=jnp.float32)
            for dd in range(3)]
    dp = (xo[0] * pool[0] + xo[1] * pool[1] + xo[2] * pool[2]
          ).reshape(OUT_CH, TS_V, NSAMPLE)
    m = jnp.max(dp, axis=2, keepdims=True)
    kio = jax.lax.broadcasted_iota(jnp.int32, (OUT_CH, TS_V, NSAMPLE), 2)
    ksel = jnp.min(jnp.where(dp == m, kio, NSAMPLE), axis=2, keepdims=True)
    onehot = kio == ksel
    outs = [jnp.sum(jnp.where(onehot, xo[dd].reshape(OUT_CH, TS_V, NSAMPLE),
                              0.0), axis=2)
            for dd in range(3)]
    y = jnp.concatenate([o[:, None, :] for o in outs], axis=1)
    out_ref[0] = y.reshape(3 * OUT_CH, TS_V)


def _vn_pallas(g48, new_xyz, W_feat, W_dir, W_pool):
    S = NPOINT
    f = jnp.arange(3 * IN_CH)
    wfd = jnp.stack([W_feat[:, f // 3] * (f % 3 == dd) for dd in range(3)])
    wdd = jnp.stack([W_dir[:, f // 3] * (f % 3 == dd) for dd in range(3)])
    wfd = jnp.pad(wfd, ((0, 0), (0, 0), (0, NFP - 3 * IN_CH)))
    wdd = jnp.pad(wdd, ((0, 0), (0, 0), (0, NFP - 3 * IN_CH)))
    return pl.pallas_call(
        _vn_body,
        grid=(B, S // TS_V),
        in_specs=[
            pl.BlockSpec((1, TS_V * NSAMPLE, NFP), lambda b, s: (b, s, 0)),
            pl.BlockSpec((1, TS_V, 3), lambda b, s: (b, s, 0)),
            pl.BlockSpec((3, OUT_CH, NFP), lambda b, s: (0, 0, 0)),
            pl.BlockSpec((3, OUT_CH, NFP), lambda b, s: (0, 0, 0)),
            pl.BlockSpec((OUT_CH, OUT_CH), lambda b, s: (0, 0)),
        ],
        out_specs=pl.BlockSpec((1, 3 * OUT_CH, TS_V), lambda b, s: (b, 0, s)),
        out_shape=jax.ShapeDtypeStruct((B, 3 * OUT_CH, S), jnp.float32),
    )(g48, new_xyz, wfd, wdd, W_pool)


def kernel(xyz, points, W_feat, W_dir, W_pool):
    xyz_t = xyz.transpose(0, 2, 1)
    pts_t = points.transpose(0, 2, 1)
    fps_idx = _fps_pallas(xyz)
    new_xyz = jax.vmap(lambda a, i: a[i])(xyz_t, fps_idx)
    idx = _knn_pallas(new_xyz, xyz)
    tbl = jnp.pad(jnp.concatenate([xyz_t, pts_t], axis=2),
                  ((0, 0), (0, 0), (0, NFP - NFEAT))).reshape(B * N, NFP)
    idx_g = (idx.reshape(B, NPOINT * NSAMPLE)
             + jnp.arange(B, dtype=jnp.int32)[:, None] * N)
    g48 = _sc_gather(tbl, idx_g.reshape(M_G // CH_G, CH_G))
    new_points = _vn_pallas(g48.reshape(B, NPOINT * NSAMPLE, NFP),
                            new_xyz, W_feat, W_dir, W_pool)
    return new_xyz.transpose(0, 2, 1), new_points, fps_idx
